# baseline (device time: 61581 ns/iter reference)
import functools

import jax
import jax.numpy as jnp
from jax import lax
from jax.experimental import pallas as pl
from jax.experimental.pallas import tpu as pltpu

N_DEV = 8


def _ring_map(k):
    return jnp.where(k < 4, k, 11 - k)


def kernel(x, W1, W2):
    m, kdim = x.shape
    _, h_per = W1.shape
    n = W2.shape[1]

    def body(x_ref, w1_ref, w2_ref, out_ref, comm_ref, send_sems, recv_sems):
        my_pos = lax.axis_index("i")
        r = _ring_map(my_pos)
        succ = _ring_map((r + 1) % N_DEV)
        pred = _ring_map((r + N_DEV - 1) % N_DEV)

        barrier_sem = pltpu.get_barrier_semaphore()
        pl.semaphore_signal(
            barrier_sem, inc=1, device_id=(succ,),
            device_id_type=pl.DeviceIdType.MESH,
        )
        pl.semaphore_signal(
            barrier_sem, inc=1, device_id=(pred,),
            device_id_type=pl.DeviceIdType.MESH,
        )
        pl.semaphore_wait(barrier_sem, 2)

        xb = x_ref[...].astype(jnp.bfloat16)
        w1b = w1_ref[...].astype(jnp.bfloat16)
        h = jnp.dot(xb, w1b, preferred_element_type=jnp.float32)
        h = jnp.maximum(h, 0.0).astype(jnp.bfloat16)
        w2b = w2_ref[...].astype(jnp.bfloat16)
        partial = jnp.dot(h, w2b, preferred_element_type=jnp.float32)

        comm_ref[0, :, :] = partial.astype(jnp.bfloat16)
        acc = partial

        for hop in range(N_DEV - 1):
            rdma = pltpu.make_async_remote_copy(
                src_ref=comm_ref.at[hop],
                dst_ref=comm_ref.at[hop + 1],
                send_sem=send_sems.at[hop],
                recv_sem=recv_sems.at[hop],
                device_id=(succ,),
                device_id_type=pl.DeviceIdType.MESH,
            )
            rdma.start()
            rdma.wait()
            acc = acc + comm_ref[hop + 1, :, :].astype(jnp.float32)

        out_ref[...] = acc

    return pl.pallas_call(
        body,
        out_shape=jax.ShapeDtypeStruct((m, n), jnp.float32),
        in_specs=[
            pl.BlockSpec(memory_space=pltpu.VMEM),
            pl.BlockSpec(memory_space=pltpu.VMEM),
            pl.BlockSpec(memory_space=pltpu.VMEM),
        ],
        out_specs=pl.BlockSpec(memory_space=pltpu.VMEM),
        scratch_shapes=[
            pltpu.VMEM((N_DEV, m, n), jnp.bfloat16),
            pltpu.SemaphoreType.DMA((N_DEV - 1,)),
            pltpu.SemaphoreType.DMA((N_DEV - 1,)),
        ],
        compiler_params=pltpu.CompilerParams(collective_id=0),
    )(x, W1, W2)


# device time: 21812 ns/iter; 2.8233x vs baseline; 2.8233x over previous
import jax
import jax.numpy as jnp
from jax import lax
from jax.experimental import pallas as pl
from jax.experimental.pallas import tpu as pltpu

N_DEV = 8


def kernel(x, W1, W2):
    m, kdim = x.shape
    _, h_per = W1.shape
    n = W2.shape[1]
    rows = m // N_DEV

    def body(x_ref, w1_ref, w2_ref, out_ref,
             p_ref, m_ref, recvbuf,
             rs_send_sems, rs_recv_sems, ag_send_sems, ag_recv_sems):
        mp = lax.axis_index("i")
        mp_off = mp * rows

        barrier_sem = pltpu.get_barrier_semaphore()
        for q in range(N_DEV):
            @pl.when(q != mp)
            def _():
                pl.semaphore_signal(
                    barrier_sem, inc=1, device_id=(q,),
                    device_id_type=pl.DeviceIdType.MESH,
                )
        pl.semaphore_wait(barrier_sem, N_DEV - 1)

        xb = x_ref[...].astype(jnp.bfloat16)
        w1b = w1_ref[...].astype(jnp.bfloat16)
        h = jnp.dot(xb, w1b, preferred_element_type=jnp.float32)
        h = jnp.maximum(h, 0.0).astype(jnp.bfloat16)
        w2b = w2_ref[...].astype(jnp.bfloat16)
        partial = jnp.dot(h, w2b, preferred_element_type=jnp.float32)
        p_ref[...] = partial.astype(jnp.bfloat16)

        rs_send = []
        rs_wait = []
        for q in range(N_DEV):
            rs_send.append(pltpu.make_async_remote_copy(
                src_ref=p_ref.at[pl.ds(q * rows, rows)],
                dst_ref=recvbuf.at[mp],
                send_sem=rs_send_sems.at[q],
                recv_sem=rs_recv_sems.at[mp],
                device_id=(q,),
                device_id_type=pl.DeviceIdType.MESH,
            ))
            rs_wait.append(pltpu.make_async_remote_copy(
                src_ref=recvbuf.at[q],
                dst_ref=recvbuf.at[q],
                send_sem=rs_send_sems.at[q],
                recv_sem=rs_recv_sems.at[q],
                device_id=(q,),
                device_id_type=pl.DeviceIdType.MESH,
            ))
        for q in range(N_DEV):
            @pl.when(q != mp)
            def _():
                rs_send[q].start()
        recvbuf[mp] = p_ref[pl.ds(mp_off, rows), :]
        for q in range(N_DEV):
            @pl.when(q != mp)
            def _():
                rs_wait[q].wait_recv()

        reduced = jnp.sum(recvbuf[...].astype(jnp.float32), axis=0)
        m_ref[pl.ds(mp_off, rows), :] = reduced.astype(jnp.bfloat16)

        ag_send = []
        ag_wait = []
        for q in range(N_DEV):
            ag_send.append(pltpu.make_async_remote_copy(
                src_ref=m_ref.at[pl.ds(mp_off, rows)],
                dst_ref=m_ref.at[pl.ds(mp_off, rows)],
                send_sem=ag_send_sems.at[q],
                recv_sem=ag_recv_sems.at[mp],
                device_id=(q,),
                device_id_type=pl.DeviceIdType.MESH,
            ))
            ag_wait.append(pltpu.make_async_remote_copy(
                src_ref=m_ref.at[pl.ds(q * rows, rows)],
                dst_ref=m_ref.at[pl.ds(q * rows, rows)],
                send_sem=ag_send_sems.at[q],
                recv_sem=ag_recv_sems.at[q],
                device_id=(q,),
                device_id_type=pl.DeviceIdType.MESH,
            ))
        for q in range(N_DEV):
            @pl.when(q != mp)
            def _():
                ag_send[q].start()
        for q in range(N_DEV):
            @pl.when(q != mp)
            def _():
                ag_wait[q].wait_recv()

        out_ref[...] = m_ref[...].astype(jnp.float32)

        for q in range(N_DEV):
            @pl.when(q != mp)
            def _():
                rs_send[q].wait_send()
                ag_send[q].wait_send()

    return pl.pallas_call(
        body,
        out_shape=jax.ShapeDtypeStruct((m, n), jnp.float32),
        in_specs=[
            pl.BlockSpec(memory_space=pltpu.VMEM),
            pl.BlockSpec(memory_space=pltpu.VMEM),
            pl.BlockSpec(memory_space=pltpu.VMEM),
        ],
        out_specs=pl.BlockSpec(memory_space=pltpu.VMEM),
        scratch_shapes=[
            pltpu.VMEM((m, n), jnp.bfloat16),
            pltpu.VMEM((m, n), jnp.bfloat16),
            pltpu.VMEM((N_DEV, rows, n), jnp.bfloat16),
            pltpu.SemaphoreType.DMA((N_DEV,)),
            pltpu.SemaphoreType.DMA((N_DEV,)),
            pltpu.SemaphoreType.DMA((N_DEV,)),
            pltpu.SemaphoreType.DMA((N_DEV,)),
        ],
        compiler_params=pltpu.CompilerParams(collective_id=0),
    )(x, W1, W2)


# device time: 20505 ns/iter; 3.0032x vs baseline; 1.0637x over previous
import jax
import jax.numpy as jnp
from jax import lax
from jax.experimental import pallas as pl
from jax.experimental.pallas import tpu as pltpu

N_DEV = 8


def kernel(x, W1, W2):
    m, kdim = x.shape
    _, h_per = W1.shape
    n = W2.shape[1]
    rows = m // N_DEV

    def body(x_ref, w1_ref, w2_ref, out_ref,
             p_ref, m_ref, recvbuf,
             rs_send_sems, rs_recv_sems, ag_send_sems, ag_recv_sems):
        mp = lax.axis_index("i")
        mp_off = mp * rows

        barrier_sem = pltpu.get_barrier_semaphore()
        for q in range(N_DEV):
            @pl.when(q != mp)
            def _():
                pl.semaphore_signal(
                    barrier_sem, inc=1, device_id=(q,),
                    device_id_type=pl.DeviceIdType.MESH,
                )

        w1b = w1_ref[...].astype(jnp.bfloat16)
        w2b = w2_ref[...].astype(jnp.bfloat16)

        rs_send = []
        for j in range(N_DEV):
            q = (mp + 1 + j) % N_DEV
            off = q * rows
            xb = x_ref[pl.ds(off, rows), :].astype(jnp.bfloat16)
            hb = jnp.dot(xb, w1b, preferred_element_type=jnp.float32)
            hb = jnp.maximum(hb, 0.0).astype(jnp.bfloat16)
            pblk = jnp.dot(hb, w2b, preferred_element_type=jnp.float32)
            if j < N_DEV - 1:
                p_ref[pl.ds(off, rows), :] = pblk.astype(jnp.bfloat16)
                if j == 0:
                    pl.semaphore_wait(barrier_sem, N_DEV - 1)
                desc = pltpu.make_async_remote_copy(
                    src_ref=p_ref.at[pl.ds(off, rows)],
                    dst_ref=recvbuf.at[mp],
                    send_sem=rs_send_sems.at[q],
                    recv_sem=rs_recv_sems.at[mp],
                    device_id=(q,),
                    device_id_type=pl.DeviceIdType.MESH,
                )
                desc.start()
                rs_send.append(desc)
            else:
                recvbuf[mp] = pblk.astype(jnp.bfloat16)

        for q in range(N_DEV):
            @pl.when(q != mp)
            def _():
                pltpu.make_async_remote_copy(
                    src_ref=recvbuf.at[q],
                    dst_ref=recvbuf.at[q],
                    send_sem=rs_send_sems.at[q],
                    recv_sem=rs_recv_sems.at[q],
                    device_id=(q,),
                    device_id_type=pl.DeviceIdType.MESH,
                ).wait_recv()

        reduced = jnp.sum(recvbuf[...].astype(jnp.float32), axis=0)
        m_ref[pl.ds(mp_off, rows), :] = reduced.astype(jnp.bfloat16)

        ag_send = []
        for q in range(N_DEV):
            desc = pltpu.make_async_remote_copy(
                src_ref=m_ref.at[pl.ds(mp_off, rows)],
                dst_ref=m_ref.at[pl.ds(mp_off, rows)],
                send_sem=ag_send_sems.at[q],
                recv_sem=ag_recv_sems.at[mp],
                device_id=(q,),
                device_id_type=pl.DeviceIdType.MESH,
            )
            ag_send.append(desc)

            @pl.when(q != mp)
            def _():
                desc.start()
        for q in range(N_DEV):
            @pl.when(q != mp)
            def _():
                pltpu.make_async_remote_copy(
                    src_ref=m_ref.at[pl.ds(q * rows, rows)],
                    dst_ref=m_ref.at[pl.ds(q * rows, rows)],
                    send_sem=ag_send_sems.at[q],
                    recv_sem=ag_recv_sems.at[q],
                    device_id=(q,),
                    device_id_type=pl.DeviceIdType.MESH,
                ).wait_recv()

        out_ref[...] = m_ref[...].astype(jnp.float32)

        for desc in rs_send:
            desc.wait_send()
        for q in range(N_DEV):
            @pl.when(q != mp)
            def _():
                ag_send[q].wait_send()

    return pl.pallas_call(
        body,
        out_shape=jax.ShapeDtypeStruct((m, n), jnp.float32),
        in_specs=[
            pl.BlockSpec(memory_space=pltpu.VMEM),
            pl.BlockSpec(memory_space=pltpu.VMEM),
            pl.BlockSpec(memory_space=pltpu.VMEM),
        ],
        out_specs=pl.BlockSpec(memory_space=pltpu.VMEM),
        scratch_shapes=[
            pltpu.VMEM((m, n), jnp.bfloat16),
            pltpu.VMEM((m, n), jnp.bfloat16),
            pltpu.VMEM((N_DEV, rows, n), jnp.bfloat16),
            pltpu.SemaphoreType.DMA((N_DEV,)),
            pltpu.SemaphoreType.DMA((N_DEV,)),
            pltpu.SemaphoreType.DMA((N_DEV,)),
            pltpu.SemaphoreType.DMA((N_DEV,)),
        ],
        compiler_params=pltpu.CompilerParams(collective_id=0),
    )(x, W1, W2)
